# Initial kernel scaffold; baseline (speedup 1.0000x reference)
import jax
import jax.numpy as jnp
from jax.experimental import pallas as pl


def kernel(x, Wg, W1, b1, W2, b2):
    raise NotImplementedError("write your pallas kernel here")


# speedup vs baseline: 1.1699x; 1.1699x over previous
import jax
import jax.numpy as jnp
from jax.experimental import pallas as pl
from jax.experimental.pallas import tpu as pltpu

D_MODEL = 1024
D_FF = 4096
E = 8
TILE = 1024


def _gate_body(x_ref, wg_ref, idx_ref):
    logits = jnp.dot(x_ref[...], wg_ref[...],
                     preferred_element_type=jnp.float32)
    idx = jnp.argmax(logits, axis=1).astype(jnp.int32)
    idx_ref[...] = jnp.broadcast_to(idx[:, None], (1, 1))


def _ffn_body(idx_ref, x_ref, w1_ref, b1_ref, w2_ref, b2_ref, o_ref):
    j = pl.program_id(0)
    h = jnp.dot(x_ref[...], w1_ref[0],
                preferred_element_type=jnp.float32) + b1_ref[0]
    h = jax.nn.gelu(h)
    contrib = jnp.dot(h, w2_ref[0], preferred_element_type=jnp.float32)

    @pl.when(j == 0)
    def _():
        o_ref[...] = b2_ref[0] + contrib

    @pl.when(j != 0)
    def _():
        o_ref[...] += contrib


def kernel(x, Wg, W1, b1, W2, b2):
    idx = pl.pallas_call(
        _gate_body,
        out_shape=jax.ShapeDtypeStruct((1, 1), jnp.int32),
    )(x, Wg)
    idx = idx.reshape((1,))

    grid_spec = pltpu.PrefetchScalarGridSpec(
        num_scalar_prefetch=1,
        grid=(D_FF // TILE,),
        in_specs=[
            pl.BlockSpec((1, D_MODEL), lambda j, idx: (0, 0)),
            pl.BlockSpec((1, D_MODEL, TILE), lambda j, idx: (idx[0], 0, j)),
            pl.BlockSpec((1, 1, TILE), lambda j, idx: (idx[0], 0, j)),
            pl.BlockSpec((1, TILE, D_MODEL), lambda j, idx: (idx[0], j, 0)),
            pl.BlockSpec((1, 1, D_MODEL), lambda j, idx: (idx[0], 0, 0)),
        ],
        out_specs=pl.BlockSpec((1, D_MODEL), lambda j, idx: (0, 0)),
    )
    out = pl.pallas_call(
        _ffn_body,
        grid_spec=grid_spec,
        out_shape=jax.ShapeDtypeStruct((1, D_MODEL), jnp.float32),
    )(idx, x, W1, b1.reshape(E, 1, D_FF), W2, b2.reshape(E, 1, D_MODEL))
    return out
